# baseline (speedup 1.0000x reference)
import functools

import jax
import jax.numpy as jnp
from jax import lax
from jax.experimental import pallas as pl
from jax.experimental.pallas import tpu as pltpu
from jax.experimental.pallas import tpu_sc as plsc

N = 10000
NP = 10240
E = 640000
NC = 2
NS = 16
NW = NC * NS
E_PER_TILE = E // NW
CH = 4000
N_CHUNKS = E_PER_TILE // CH
N_PER_TILE = NP // NS
FD = 8


def _mesh():
    return plsc.VectorSubcoreMesh(core_axis_name="c", subcore_axis_name="s")


_SC_PARAMS = pltpu.CompilerParams(use_tc_tiling_on_sc=False)


@functools.partial(
    pl.kernel,
    out_type=jax.ShapeDtypeStruct((NC, NP, FD), jnp.float32),
    mesh=_mesh(),
    compiler_params=_SC_PARAMS,
    scratch_types=[
        pltpu.VMEM_SHARED((NP, FD), jnp.float32),
        pltpu.VMEM((CH,), jnp.int32),
        pltpu.VMEM((CH, FD), jnp.float32),
    ],
)
def _sc_degree(ei_hbm, ones_hbm, zeros_hbm, out_hbm, acc_sp, idx_v, ones_v):
    cid = lax.axis_index("c")
    sid = lax.axis_index("s")
    wid = cid * NS + sid
    rbase = sid * N_PER_TILE
    pltpu.sync_copy(zeros_hbm.at[pl.ds(rbase, N_PER_TILE)],
                    acc_sp.at[pl.ds(rbase, N_PER_TILE)])
    pltpu.sync_copy(ones_hbm, ones_v)
    plsc.subcore_barrier()

    def step(k, carry):
        base = wid * E_PER_TILE + k * CH
        pltpu.sync_copy(ei_hbm.at[1, pl.ds(base, CH)], idx_v)
        pltpu.sync_copy(ones_v, acc_sp.at[idx_v], add=True)
        return carry

    lax.fori_loop(0, N_CHUNKS, step, 0)
    plsc.subcore_barrier()
    pltpu.sync_copy(acc_sp.at[pl.ds(rbase, N_PER_TILE)],
                    out_hbm.at[cid, pl.ds(rbase, N_PER_TILE)])


def _make_msgpass(f):
    @functools.partial(
        pl.kernel,
        out_type=jax.ShapeDtypeStruct((NC, NP, f), jnp.float32),
        mesh=_mesh(),
        compiler_params=_SC_PARAMS,
        scratch_types=[
            pltpu.VMEM_SHARED((NP, f), jnp.float32),
            pltpu.VMEM_SHARED((NP, f), jnp.float32),
            pltpu.VMEM((2, CH), jnp.int32),
            pltpu.VMEM((CH, f), jnp.float32),
        ],
    )
    def _msgpass(g_hbm, ei_hbm, zeros_hbm, out_hbm,
                 g_sp, acc_sp, idx_v, msgs_v):
        cid = lax.axis_index("c")
        sid = lax.axis_index("s")
        wid = cid * NS + sid
        rbase = sid * N_PER_TILE
        pltpu.sync_copy(zeros_hbm.at[pl.ds(rbase, N_PER_TILE)],
                        acc_sp.at[pl.ds(rbase, N_PER_TILE)])
        pltpu.sync_copy(g_hbm.at[pl.ds(rbase, N_PER_TILE)],
                        g_sp.at[pl.ds(rbase, N_PER_TILE)])
        plsc.subcore_barrier()

        def step(k, carry):
            base = wid * E_PER_TILE + k * CH
            pltpu.sync_copy(ei_hbm.at[:, pl.ds(base, CH)], idx_v)
            pltpu.sync_copy(g_sp.at[idx_v.at[0]], msgs_v)
            pltpu.sync_copy(msgs_v, acc_sp.at[idx_v.at[1]], add=True)
            return carry

        lax.fori_loop(0, N_CHUNKS, step, 0)
        plsc.subcore_barrier()
        pltpu.sync_copy(acc_sp.at[pl.ds(rbase, N_PER_TILE)],
                        out_hbm.at[cid, pl.ds(rbase, N_PER_TILE)])

    return _msgpass


_msgpass16 = _make_msgpass(16)
_msgpass8 = _make_msgpass(8)


def _tc0_body(x_ref, w1_ref, h_ref):
    h = jnp.dot(x_ref[...], w1_ref[...], preferred_element_type=jnp.float32)
    h_ref[:N] = h
    h_ref[N:] = jnp.zeros((NP - N, 16), jnp.float32)


def _tc1_body(h_ref, degp_ref, g1_ref, dinv_ref):
    deg8 = degp_ref[0] + degp_ref[1] + 1.0
    dinv8 = lax.rsqrt(deg8)
    dinv16 = jnp.concatenate([dinv8, dinv8], axis=-1)
    g1_ref[...] = dinv16 * h_ref[...]
    dinv_ref[...] = dinv16


def _tc2_body(acc_ref, g1_ref, dinv_ref, b1_ref, w2_ref, g2_ref):
    s = acc_ref[0] + acc_ref[1] + g1_ref[...]
    h1 = jnp.maximum(dinv_ref[...] * s + b1_ref[...], 0.0)
    h2 = jnp.dot(h1, w2_ref[...], preferred_element_type=jnp.float32)
    g2_ref[...] = dinv_ref[:, :8] * h2


def _tc3_body(acc_ref, g2_ref, dinv_ref, b2_ref, wfcs_ref, bfc_ref, out_ref):
    s = acc_ref[0] + acc_ref[1] + g2_ref[...]
    h = jnp.maximum(dinv_ref[:, :8] * s + b2_ref[...], 0.0)
    prod = h[None, :, :] * wfcs_ref[...]
    sums = jnp.sum(prod, axis=(1, 2))
    out_ref[...] = sums.reshape(1, 2) + bfc_ref[...]


def kernel(x, edge_index, w1, b1, w2, b2, wfc, bfc):
    ones_d = jnp.ones((CH, FD), jnp.float32)
    zeros8 = jnp.zeros((NP, FD), jnp.float32)
    zeros16 = jnp.zeros((NP, 16), jnp.float32)

    hp = pl.pallas_call(
        _tc0_body,
        out_shape=jax.ShapeDtypeStruct((NP, 16), jnp.float32),
    )(x, w1)
    degp = _sc_degree(edge_index, ones_d, zeros8)

    g1, dinv16 = pl.pallas_call(
        _tc1_body,
        out_shape=(jax.ShapeDtypeStruct((NP, 16), jnp.float32),
                   jax.ShapeDtypeStruct((NP, 16), jnp.float32)),
    )(hp, degp)

    acc1 = _msgpass16(g1, edge_index, zeros16)

    g2 = pl.pallas_call(
        _tc2_body,
        out_shape=jax.ShapeDtypeStruct((NP, 8), jnp.float32),
    )(acc1, g1, dinv16, b1.reshape(1, 16), w2)

    acc2 = _msgpass8(g2, edge_index, zeros8)

    wfcs = jnp.pad(wfc.reshape(N, 8, 2).transpose(2, 0, 1),
                   ((0, 0), (0, NP - N), (0, 0)))
    out = pl.pallas_call(
        _tc3_body,
        out_shape=jax.ShapeDtypeStruct((1, 2), jnp.float32),
    )(acc2, g2, dinv16, b2.reshape(1, 8), wfcs, bfc.reshape(1, 2))
    return out


# speedup vs baseline: 1.0238x; 1.0238x over previous
import functools

import jax
import jax.numpy as jnp
from jax import lax
from jax.experimental import pallas as pl
from jax.experimental.pallas import tpu as pltpu
from jax.experimental.pallas import tpu_sc as plsc

N = 10000
NP = 10240
E = 640000
NC = 2
NS = 16
NW = NC * NS
E_PER_TILE = E // NW
CH = 4000
N_CHUNKS = E_PER_TILE // CH
N_PER_TILE = NP // NS
FD = 8


def _mesh():
    return plsc.VectorSubcoreMesh(core_axis_name="c", subcore_axis_name="s")


_SC_PARAMS = pltpu.CompilerParams(use_tc_tiling_on_sc=False)


@functools.partial(
    pl.kernel,
    out_type=jax.ShapeDtypeStruct((NC, NP, FD), jnp.float32),
    mesh=_mesh(),
    compiler_params=_SC_PARAMS,
    scratch_types=[
        pltpu.VMEM_SHARED((NP, FD), jnp.float32),
        pltpu.VMEM((CH,), jnp.int32),
        pltpu.VMEM((CH, FD), jnp.float32),
    ],
)
def _sc_degree(ei_hbm, ones_hbm, zeros_hbm, out_hbm, acc_sp, idx_v, ones_v):
    cid = lax.axis_index("c")
    sid = lax.axis_index("s")
    wid = cid * NS + sid
    rbase = sid * N_PER_TILE
    pltpu.sync_copy(zeros_hbm.at[pl.ds(rbase, N_PER_TILE)],
                    acc_sp.at[pl.ds(rbase, N_PER_TILE)])
    pltpu.sync_copy(ones_hbm, ones_v)
    plsc.subcore_barrier()

    def step(k, carry):
        base = wid * E_PER_TILE + k * CH
        pltpu.sync_copy(ei_hbm.at[1, pl.ds(base, CH)], idx_v)
        pltpu.sync_copy(ones_v, acc_sp.at[idx_v], add=True)
        return carry

    lax.fori_loop(0, N_CHUNKS, step, 0)
    plsc.subcore_barrier()
    pltpu.sync_copy(acc_sp.at[pl.ds(rbase, N_PER_TILE)],
                    out_hbm.at[cid, pl.ds(rbase, N_PER_TILE)])


def _make_msgpass(f):
    ch = 2000 if f == 16 else 5000
    nch = E_PER_TILE // ch

    @functools.partial(
        pl.kernel,
        out_type=jax.ShapeDtypeStruct((NC, NP, f), jnp.float32),
        mesh=_mesh(),
        compiler_params=_SC_PARAMS,
        scratch_types=[
            pltpu.VMEM_SHARED((NP, f), jnp.float32),
            pltpu.VMEM_SHARED((NP, f), jnp.float32),
            pltpu.VMEM((2, ch), jnp.int32),
            pltpu.VMEM((2, ch), jnp.int32),
            pltpu.VMEM((ch, f), jnp.float32),
            pltpu.VMEM((ch, f), jnp.float32),
            pltpu.SemaphoreType.DMA,
            pltpu.SemaphoreType.DMA,
            pltpu.SemaphoreType.DMA,
            pltpu.SemaphoreType.DMA,
        ],
    )
    def _msgpass(g_hbm, ei_hbm, zeros_hbm, out_hbm,
                 g_sp, acc_sp, idx0, idx1, msgs0, msgs1,
                 gsem0, gsem1, ssem0, ssem1):
        cid = lax.axis_index("c")
        sid = lax.axis_index("s")
        wid = cid * NS + sid
        rbase = sid * N_PER_TILE
        ebase = wid * E_PER_TILE
        pltpu.sync_copy(zeros_hbm.at[pl.ds(rbase, N_PER_TILE)],
                        acc_sp.at[pl.ds(rbase, N_PER_TILE)])
        pltpu.sync_copy(g_hbm.at[pl.ds(rbase, N_PER_TILE)],
                        g_sp.at[pl.ds(rbase, N_PER_TILE)])
        plsc.subcore_barrier()

        idx = [idx0, idx1]
        msgs = [msgs0, msgs1]
        gsem = [gsem0, gsem1]
        ssem = [ssem0, ssem1]
        ghandle = [None, None]
        shandle = [None, None]

        pltpu.sync_copy(ei_hbm.at[:, pl.ds(ebase, ch)], idx[0])
        ghandle[0] = pltpu.async_copy(g_sp.at[idx[0].at[0]], msgs[0], gsem[0])
        for k in range(nch):
            cur = k % 2
            nxt = (k + 1) % 2
            if k + 1 < nch:
                if k >= 1:
                    shandle[nxt].wait()
                pltpu.sync_copy(
                    ei_hbm.at[:, pl.ds(ebase + (k + 1) * ch, ch)], idx[nxt])
            ghandle[cur].wait()
            if k + 1 < nch:
                ghandle[nxt] = pltpu.async_copy(
                    g_sp.at[idx[nxt].at[0]], msgs[nxt], gsem[nxt])
            shandle[cur] = pltpu.async_copy(
                msgs[cur], acc_sp.at[idx[cur].at[1]], ssem[cur], add=True)
        shandle[(nch - 1) % 2].wait()
        if nch >= 2:
            shandle[(nch - 2) % 2].wait()

        plsc.subcore_barrier()
        pltpu.sync_copy(acc_sp.at[pl.ds(rbase, N_PER_TILE)],
                        out_hbm.at[cid, pl.ds(rbase, N_PER_TILE)])

    return _msgpass


_msgpass16 = _make_msgpass(16)
_msgpass8 = _make_msgpass(8)


def _tc1_body(x_ref, w1_ref, degp_ref, g1_ref, dinv_ref):
    deg8 = degp_ref[0] + degp_ref[1] + 1.0
    dinv8 = lax.rsqrt(deg8)
    dinv16 = jnp.concatenate([dinv8, dinv8], axis=-1)
    h = jnp.dot(x_ref[...], w1_ref[...], preferred_element_type=jnp.float32)
    g1_ref[:N] = dinv16[:N] * h
    g1_ref[N:] = jnp.zeros((NP - N, 16), jnp.float32)
    dinv_ref[...] = dinv16


def _tc2_body(acc_ref, g1_ref, dinv_ref, b1_ref, w2_ref, g2_ref):
    s = acc_ref[0] + acc_ref[1] + g1_ref[...]
    h1 = jnp.maximum(dinv_ref[...] * s + b1_ref[...], 0.0)
    h2 = jnp.dot(h1, w2_ref[...], preferred_element_type=jnp.float32)
    g2_ref[...] = dinv_ref[:, :8] * h2


def _tc3_body(acc_ref, g2_ref, dinv_ref, b2_ref, wfcs_ref, bfc_ref, out_ref):
    s = acc_ref[0] + acc_ref[1] + g2_ref[...]
    h = jnp.maximum(dinv_ref[:, :8] * s + b2_ref[...], 0.0)
    prod = h[None, :, :] * wfcs_ref[...]
    sums = jnp.sum(prod, axis=(1, 2))
    out_ref[...] = sums.reshape(1, 2) + bfc_ref[...]


def kernel(x, edge_index, w1, b1, w2, b2, wfc, bfc):
    ones_d = jnp.ones((CH, FD), jnp.float32)
    zeros8 = jnp.zeros((NP, FD), jnp.float32)
    zeros16 = jnp.zeros((NP, 16), jnp.float32)

    degp = _sc_degree(edge_index, ones_d, zeros8)

    g1, dinv16 = pl.pallas_call(
        _tc1_body,
        out_shape=(jax.ShapeDtypeStruct((NP, 16), jnp.float32),
                   jax.ShapeDtypeStruct((NP, 16), jnp.float32)),
    )(x, w1, degp)

    acc1 = _msgpass16(g1, edge_index, zeros16)

    g2 = pl.pallas_call(
        _tc2_body,
        out_shape=jax.ShapeDtypeStruct((NP, 8), jnp.float32),
    )(acc1, g1, dinv16, b1.reshape(1, 16), w2)

    acc2 = _msgpass8(g2, edge_index, zeros8)

    wfcs = jnp.pad(wfc.reshape(N, 8, 2).transpose(2, 0, 1),
                   ((0, 0), (0, NP - N), (0, 0)))
    out = pl.pallas_call(
        _tc3_body,
        out_shape=jax.ShapeDtypeStruct((1, 2), jnp.float32),
    )(acc2, g2, dinv16, b2.reshape(1, 8), wfcs, bfc.reshape(1, 2))
    return out
